# baseline (speedup 1.0000x reference)
import jax
import jax.numpy as jnp
from jax import lax
from jax.experimental import pallas as pl
from jax.experimental.pallas import tpu as pltpu, tpu_sc as plsc

N_WORKERS = 32
CHUNK = 1024
W = 128
GROUPS = CHUNK // 16
BLOCKS = CHUNK // 128
RPC = CHUNK // W + 1


def _body(xs_hbm, ys_hbm, zs_hbm, tab_hbm, co_hbm, do_hbm,
          xin_a, cidx_a, pos_a, out_a, dov_a,
          xin_b, cidx_b, pos_b, out_b, dov_b,
          rows_v, sem_a, sem_b, sem_ia, sem_ib):
    wid = lax.axis_index("s") * 2 + lax.axis_index("c")
    n_pts = xs_hbm.shape[0]
    per_worker = n_pts // N_WORKERS
    n_chunks = per_worker // CHUNK

    ii = lax.iota(jnp.int32, 16)
    half16 = jnp.full((16,), 0.5, jnp.float32)
    zerof16 = jnp.zeros((16,), jnp.float32)
    zero16 = jnp.zeros((16,), jnp.int32)
    one16 = zero16 + 1
    two16 = zero16 + 2

    def fire_in(k, xin, sem):
        base = wid * per_worker + k * CHUNK
        pltpu.async_copy(xs_hbm.at[pl.ds(base, CHUNK)], xin.at[0], sem)
        pltpu.async_copy(ys_hbm.at[pl.ds(base, CHUNK)], xin.at[1], sem)
        pltpu.async_copy(zs_hbm.at[pl.ds(base, CHUNK)], xin.at[2], sem)

    def drain_in(k, xin, sem):
        base = wid * per_worker + k * CHUNK
        pltpu.make_async_copy(xs_hbm.at[pl.ds(base, CHUNK)], xin.at[0], sem).wait()
        pltpu.make_async_copy(ys_hbm.at[pl.ds(base, CHUNK)], xin.at[1], sem).wait()
        pltpu.make_async_copy(zs_hbm.at[pl.ds(base, CHUNK)], xin.at[2], sem).wait()

    def stage_a(k, xin, sem_in, kn, xin_n, sem_in_n,
                cidx_v, pos_v, out_v, dov_v):
        drain_in(k, xin, sem_in)
        fire_in(kn, xin_n, sem_in_n)

        def half_group(g, base_cnt):
            o = g * 16
            x = xin[0, pl.ds(o, 16)]
            y = xin[1, pl.ds(o, 16)]
            z = xin[2, pl.ds(o, 16)]
            ux = jnp.clip(x * 128.0 + 128.0, 0.0, 255.0).astype(jnp.int32)
            uy = jnp.clip(y * 128.0 + 128.0, 0.0, 255.0).astype(jnp.int32)
            uz = jnp.clip(z * 128.0 + 128.0, 0.0, 255.0).astype(jnp.int32)
            zt = uz >> 7
            zl = uz & 127
            g0 = ((((ux << 8) | uy) << 1 | zt) << 5) | (zl >> 4)
            m = jnp.maximum(jnp.maximum(jnp.abs(x), jnp.abs(y)), jnp.abs(z))
            keep = m < 1.0
            tgt = base_cnt + plsc.cumsum(keep.astype(jnp.int32)) - 1
            meta = ((ii + o) << 4) | (zl & 15)
            return g0, keep, tgt, meta

        def emit_group(g0, keep, tgt, meta):
            row = tgt >> 7
            col = tgt & 127
            plsc.store_scatter(cidx_v, [row, col], g0, mask=keep)
            plsc.store_scatter(cidx_v, [RPC + row, col], g0 + 8, mask=keep)
            plsc.store_scatter(cidx_v, [2 * RPC + row, col], g0 + 16, mask=keep)
            plsc.store_scatter(cidx_v, [3 * RPC + row, col], g0 + 24, mask=keep)
            plsc.store_scatter(pos_v, [tgt], meta, mask=keep)

        def index_body(h, mcount):
            ga = 2 * h
            g0a, keepa, tgta, metaa = half_group(ga, mcount)
            cnta = tgta[15] + 1
            g0b, keepb, tgtb, metab = half_group(ga + 1, cnta)
            emit_group(g0a, keepa, tgta, metaa)
            emit_group(g0b, keepb, tgtb, metab)
            nt = h // 4
            c0 = (h % 4) * 32
            out_v[nt, 0, pl.ds(c0, 16)] = half16
            out_v[nt, 0, pl.ds(c0 + 16, 16)] = half16
            out_v[nt, 1, pl.ds(c0, 16)] = half16
            out_v[nt, 1, pl.ds(c0 + 16, 16)] = half16
            out_v[nt, 2, pl.ds(c0, 16)] = half16
            out_v[nt, 2, pl.ds(c0 + 16, 16)] = half16
            dov_v[pl.ds(ga * 16, 16)] = zerof16
            dov_v[pl.ds(ga * 16 + 16, 16)] = zerof16
            return tgtb[15] + 1

        mcount = lax.fori_loop(0, GROUPS // 2, index_body, jnp.int32(0))

        for j in range(8):
            tgt = mcount + j * 16 + ii
            row = tgt >> 7
            col = tgt & 127
            dummy = ((wid << 8) | (j * 16 + ii)) << 5
            plsc.store_scatter(cidx_v, [row, col], dummy)
            plsc.store_scatter(cidx_v, [RPC + row, col], dummy)
            plsc.store_scatter(cidx_v, [2 * RPC + row, col], dummy)
            plsc.store_scatter(cidx_v, [3 * RPC + row, col], dummy)
        return mcount

    def fire(mcount, cidx_v, sem):
        def fire_body(r, _):
            for c in range(4):
                pltpu.async_copy(tab_hbm.at[cidx_v.at[c * RPC + r]],
                                 rows_v.at[c * RPC + r], sem)
            return 0
        lax.fori_loop(0, (mcount + 127) >> 7, fire_body, 0)

    def drain(mcount, cidx_v, sem):
        def drain_body(r, _):
            for c in range(4):
                pltpu.make_async_copy(tab_hbm.at[cidx_v.at[c * RPC + r]],
                                      rows_v.at[c * RPC + r], sem).wait()
            return 0
        lax.fori_loop(0, (mcount + 127) >> 7, drain_body, 0)

    def stage_c(k, mcount, pos_v, out_v, dov_v):
        def post_body(t, _):
            o = t * 16
            s = o + ii
            active = s < mcount
            meta = pos_v[pl.ds(o, 16)]
            p = meta >> 4
            sub = meta & 15
            row = s >> 7
            col = s & 127
            pt = p >> 7
            pc = p & 127
            vr = plsc.load_gather(rows_v, [row, col, sub], mask=active)
            vg = plsc.load_gather(rows_v, [RPC + row, col, sub], mask=active)
            vb = plsc.load_gather(rows_v, [2 * RPC + row, col, sub], mask=active)
            vd = plsc.load_gather(rows_v, [3 * RPC + row, col, sub], mask=active)
            plsc.store_scatter(out_v, [pt, zero16, pc],
                               1.0 / (1.0 + jnp.exp(-vr)), mask=active)
            plsc.store_scatter(out_v, [pt, one16, pc],
                               1.0 / (1.0 + jnp.exp(-vg)), mask=active)
            plsc.store_scatter(out_v, [pt, two16, pc],
                               1.0 / (1.0 + jnp.exp(-vb)), mask=active)
            plsc.store_scatter(dov_v, [p],
                               jnp.maximum(vd * 10.0, 0.0), mask=active)
            return 0

        lax.fori_loop(0, (mcount + 15) >> 4, post_body, 0)
        base = wid * per_worker + k * CHUNK
        pltpu.sync_copy(out_v, co_hbm.at[pl.ds(base // 128, BLOCKS)])
        pltpu.sync_copy(dov_v, do_hbm.at[pl.ds(base, CHUNK)])

    def two_chunks(j, m_prev):
        k0 = 2 * j
        m0 = stage_a(k0, xin_a, sem_ia, k0 + 1, xin_b, sem_ib,
                     cidx_a, pos_a, out_a, dov_a)

        @pl.when(j > 0)
        def _():
            drain(m_prev, cidx_b, sem_b)
            stage_c(k0 - 1, m_prev, pos_b, out_b, dov_b)

        fire(m0, cidx_a, sem_a)
        m1 = stage_a(k0 + 1, xin_b, sem_ib, (k0 + 2) % n_chunks, xin_a, sem_ia,
                     cidx_b, pos_b, out_b, dov_b)
        drain(m0, cidx_a, sem_a)
        stage_c(k0, m0, pos_a, out_a, dov_a)
        fire(m1, cidx_b, sem_b)
        return m1

    fire_in(0, xin_a, sem_ia)
    m_last = lax.fori_loop(0, n_chunks // 2, two_chunks, jnp.int32(0))
    drain(m_last, cidx_b, sem_b)
    stage_c(n_chunks - 1, m_last, pos_b, out_b, dov_b)
    drain_in(0, xin_a, sem_ia)


@jax.jit
def _sc_voxels(xs, ys, zs, tab):
    n_pts = xs.shape[0]
    mesh = plsc.VectorSubcoreMesh(core_axis_name="c", subcore_axis_name="s")
    buf = lambda dt, *shape: pltpu.VMEM(tuple(shape), dt)
    pair = lambda: (
        buf(jnp.float32, 3, CHUNK),
        buf(jnp.int32, 4 * RPC, W),
        buf(jnp.int32, CHUNK + 128),
        buf(jnp.float32, BLOCKS, 4, 128),
        buf(jnp.float32, CHUNK),
    )
    return pl.kernel(
        _body,
        out_type=(
            jax.ShapeDtypeStruct((n_pts // 128, 4, 128), jnp.float32),
            jax.ShapeDtypeStruct((n_pts,), jnp.float32),
        ),
        mesh=mesh,
        compiler_params=pltpu.CompilerParams(
            needs_layout_passes=False, use_tc_tiling_on_sc=False),
        scratch_types=[
            *pair(), *pair(),
            buf(jnp.float32, 4 * RPC, W, 16),
            pltpu.SemaphoreType.DMA, pltpu.SemaphoreType.DMA,
            pltpu.SemaphoreType.DMA, pltpu.SemaphoreType.DMA,
        ],
    )(xs, ys, zs, tab)


def kernel(xyz, voxels):
    n = xyz.shape[0]
    tab = (voxels.reshape(256, 256, 2, 128, 4)
           .transpose(0, 1, 2, 4, 3).reshape(1 << 22, 16))
    co, d = _sc_voxels(xyz[:, 0], xyz[:, 1], xyz[:, 2], tab)
    colors = co.transpose(0, 2, 1).reshape(n, 4)[:, :3]
    return colors, d
